# baseline (speedup 1.0000x reference)
import jax
import jax.numpy as jnp
from jax.experimental import pallas as pl
from jax.experimental.pallas import tpu as pltpu


def _conv2d(x, w, b, padding=1):
    out = jax.lax.conv_general_dilated(
        x, w, window_strides=(1, 1),
        padding=((padding, padding), (padding, padding)),
        dimension_numbers=('NCHW', 'OIHW', 'NCHW'))
    return out + b[None, :, None, None]


def _maxpool_same(x, k=3):
    pad = (k - 1) // 2
    return jax.lax.reduce_window(
        x, -jnp.inf, jax.lax.max,
        (1, 1, k, k), (1, 1, 1, 1),
        ((0, 0), (0, 0), (pad, pad), (pad, pad)))


def _nms_body(hm_ref, hmax_ref, out_ref):
    hm = hm_ref[...]
    hmax = hmax_ref[...]
    out_ref[...] = jnp.where(hmax == hm, hm, 0.0)


def kernel(x, W_shared, b_shared, W_hm, b_hm):
    obj_num = 500
    B, C, H, Wd = x.shape
    feat = jax.nn.relu(_conv2d(x, W_shared, b_shared))
    hm = jax.nn.sigmoid(_conv2d(feat, W_hm, b_hm))
    hmax = _maxpool_same(hm, 3)
    num_cls = hm.shape[1]
    hm2 = hm.reshape(num_cls * H, Wd)
    hmax2 = hmax.reshape(num_cls * H, Wd)
    scores_map = pl.pallas_call(
        _nms_body,
        out_shape=jax.ShapeDtypeStruct((num_cls * H, Wd), jnp.float32),
    )(hm2, hmax2)
    flat = scores_map.reshape(B, num_cls * H * Wd)
    scores, inds = jax.lax.top_k(flat, obj_num)
    clses = (inds // (H * Wd)).astype(jnp.int32)
    pos = inds % (H * Wd)
    ys = (pos // Wd).astype(jnp.float32)
    xs = (pos % Wd).astype(jnp.float32)
    feat_flat = feat.reshape(B, C, H * Wd)
    ct_feat = jnp.take_along_axis(feat_flat, pos[:, None, :], axis=2)
    ct_feat = jnp.transpose(ct_feat, (0, 2, 1))
    return ct_feat, scores, xs, ys, clses


# speedup vs baseline: 1.4333x; 1.4333x over previous
import jax
import jax.numpy as jnp
from jax.experimental import pallas as pl
from jax.experimental.pallas import tpu as pltpu

H = 180
W = 180
C = 256
WP = 192
NUM_CLS = 10
OBJ_NUM = 500
R = 30
NB = H // R
NROWS = (R + 2) * WP


def _conv_body(x3_hbm, w9_ref, b_ref, out_ref, S, V0, V2, sems):
    i = pl.program_id(0)

    def start(block, slot):
        pltpu.make_async_copy(
            x3_hbm.at[pl.ds(block * R * WP, NROWS + 16)],
            S.at[slot], sems.at[slot]).start()

    def wait(block, slot):
        pltpu.make_async_copy(
            x3_hbm.at[pl.ds(block * R * WP, NROWS + 16)],
            S.at[slot], sems.at[slot]).wait()

    @pl.when(i == 0)
    def _():
        start(0, 0)

    @pl.when(i + 1 < NB)
    def _():
        start(i + 1, (i + 1) % 2)

    wait(i, i % 2)
    slot = i % 2
    V0[...] = S[slot, pl.ds(7, NROWS), :]
    V2[...] = S[slot, pl.ds(9, NROWS), :]

    def lhs(dy, dx):
        if dx == 0:
            return V0[pl.ds(dy * WP, R * WP), :]
        if dx == 2:
            return V2[pl.ds(dy * WP, R * WP), :]
        return S[slot, pl.ds(dy * WP + 8, R * WP), :]

    acc = jnp.dot(lhs(0, 0), w9_ref[0], preferred_element_type=jnp.float32)
    for k in range(1, 9):
        dy, dx = divmod(k, 3)
        acc = acc + jnp.dot(lhs(dy, dx), w9_ref[k],
                            preferred_element_type=jnp.float32)
    col = jax.lax.broadcasted_iota(jnp.int32, (R * WP, C), 0) % WP
    valid = col < W
    out_ref[...] = jnp.where(valid, jnp.maximum(acc + b_ref[0], 0.0), 0.0)


def _shared_conv(x3, w9, b2):
    return pl.pallas_call(
        _conv_body,
        grid=(NB,),
        in_specs=[
            pl.BlockSpec(memory_space=pltpu.MemorySpace.HBM),
            pl.BlockSpec((9, C, C), lambda i: (0, 0, 0)),
            pl.BlockSpec((1, C), lambda i: (0, 0)),
        ],
        out_specs=pl.BlockSpec((R * WP, C), lambda i: (i, 0)),
        out_shape=jax.ShapeDtypeStruct((H * WP, C), jnp.float32),
        scratch_shapes=[
            pltpu.VMEM((2, NROWS + 16, C), jnp.float32),
            pltpu.VMEM((NROWS, C), jnp.float32),
            pltpu.VMEM((NROWS, C), jnp.float32),
            pltpu.SemaphoreType.DMA((2,)),
        ],
    )(x3, w9, b2)


def kernel(x, W_shared, b_shared, W_hm, b_hm):
    xt = jnp.transpose(x[0], (1, 2, 0))
    x3 = jnp.zeros((183, WP, C), jnp.float32)
    x3 = x3.at[1:H + 1, 8:8 + W, :].set(xt).reshape(183 * WP, C)
    w9 = jnp.transpose(W_shared, (2, 3, 1, 0)).reshape(9, C, C)
    feat_wide = _shared_conv(x3, w9, b_shared[None, :])

    feat = feat_wide.reshape(H, WP, C)[:, :W, :]

    w_hm = jnp.transpose(W_hm, (2, 3, 1, 0))
    hm = jax.lax.conv_general_dilated(
        feat[None], w_hm, window_strides=(1, 1),
        padding=((1, 1), (1, 1)),
        dimension_numbers=('NHWC', 'HWIO', 'NHWC'))
    hm = jax.nn.sigmoid(hm + b_hm[None, None, None, :])
    hmax = jax.lax.reduce_window(
        hm, -jnp.inf, jax.lax.max,
        (1, 3, 3, 1), (1, 1, 1, 1),
        ((0, 0), (1, 1), (1, 1), (0, 0)))
    scores_map = jnp.where(hmax == hm, hm, 0.0)
    flat = scores_map.reshape(1, H * W * NUM_CLS)
    scores, inds = jax.lax.top_k(flat, OBJ_NUM)
    clses = (inds % NUM_CLS).astype(jnp.int32)
    pos = inds // NUM_CLS
    ys = (pos // W).astype(jnp.float32)
    xs = (pos % W).astype(jnp.float32)
    rows = pos[0] + (pos[0] // W) * (WP - W)
    ct_feat = jnp.take(feat_wide, rows, axis=0)[None]
    return ct_feat, scores, xs, ys, clses
